# Initial kernel scaffold; baseline (speedup 1.0000x reference)
import jax
import jax.numpy as jnp
from jax.experimental import pallas as pl


def kernel(inputs, tables):
    raise NotImplementedError("write your pallas kernel here")


# speedup vs baseline: 1.1198x; 1.1198x over previous
import jax
import jax.numpy as jnp
from jax import lax
from jax.experimental import pallas as pl
from jax.experimental.pallas import tpu as pltpu
from jax.experimental.pallas import tpu_sc as plsc

B = 16384
N_EMB = 26
VOCAB = 100000
EDIM = 16
N_DENSE = 13
OUT_D = N_EMB * EDIM + N_DENSE

NC = 2
NS = 16
NW = NC * NS
ROWS_W = B // NW


def _body(in_hbm, tab_hbm, out_hbm, in_v, idx_v, rows_v, dense_v, sem):
    wid = lax.axis_index("s") * NC + lax.axis_index("c")
    base = wid * ROWS_W

    pltpu.sync_copy(in_hbm.at[pl.ds(base, ROWS_W)], in_v)

    def vec_dense(j, c2):
        rows = lax.iota(jnp.int32, 16) + j * 16
        for c in range(N_DENSE):
            vals = plsc.load_gather(in_v, [rows, jnp.full((16,), N_EMB + c, jnp.int32)])
            plsc.store_scatter(dense_v, [rows, jnp.full((16,), c, jnp.int32)], vals)
        return c2

    lax.fori_loop(0, ROWS_W // 16, vec_dense, 0)
    pltpu.sync_copy(
        dense_v,
        out_hbm.at[pl.ds(base, ROWS_W), pl.ds(N_EMB * EDIM, N_DENSE)],
    )

    def col(i, carry):
        def vec(j, c2):
            rows = lax.iota(jnp.int32, 16) + j * 16
            cols = jnp.full((16,), i, jnp.int32)
            vals = plsc.load_gather(in_v, [rows, cols])
            off = pl.multiple_of(j * 16, 16)
            idx_v[pl.ds(off, 16)] = vals.astype(jnp.int32) + i * VOCAB
            return c2

        lax.fori_loop(0, ROWS_W // 16, vec, 0)
        pltpu.async_copy(tab_hbm.at[idx_v], rows_v, sem).wait()
        pltpu.sync_copy(
            rows_v,
            out_hbm.at[pl.ds(base, ROWS_W), pl.ds(i * EDIM, EDIM)],
        )
        return carry

    lax.fori_loop(0, N_EMB, col, 0)


def kernel(inputs, tables):
    tab = tables.reshape(N_EMB * VOCAB, EDIM)
    mesh = plsc.VectorSubcoreMesh(core_axis_name="c", subcore_axis_name="s")
    k = pl.kernel(
        _body,
        out_type=jax.ShapeDtypeStruct((B, OUT_D), jnp.float32),
        mesh=mesh,
        scratch_types=[
            pltpu.VMEM((ROWS_W, N_EMB + N_DENSE), jnp.float32),
            pltpu.VMEM((ROWS_W,), jnp.int32),
            pltpu.VMEM((ROWS_W, EDIM), jnp.float32),
            pltpu.VMEM((ROWS_W, N_DENSE), jnp.float32),
            pltpu.SemaphoreType.DMA,
        ],
        compiler_params=pltpu.CompilerParams(
            use_tc_tiling_on_sc=False, needs_layout_passes=False
        ),
    )
    return k(inputs, tab)
